# Initial kernel scaffold; baseline (speedup 1.0000x reference)
import functools

import jax
import jax.numpy as jnp
from jax import lax
from jax.experimental import pallas as pl
from jax.experimental.pallas import tpu as pltpu
from jax.experimental.pallas import tpu_sc as plsc

B = 16384
F = 26
V = 100000
D = 32
ND = 13
OUT = 128
HIDDEN = F * D + ND

NC = 2
NS = 16
NW = NC * NS

ROWS = B * F
RPW = ROWS // NW
CHUNK = 1024
NCHUNK = RPW // CHUNK


def _sc_gather(table_flat, flat_idx):
    mesh = plsc.VectorSubcoreMesh(core_axis_name="c", subcore_axis_name="s",
                                  num_cores=NC, num_subcores=NS)

    @functools.partial(
        pl.kernel,
        out_type=jax.ShapeDtypeStruct((ROWS, D), jnp.float32),
        mesh=mesh,
        scratch_types=[
            pltpu.VMEM((RPW,), jnp.int32),
            pltpu.VMEM((CHUNK, D), jnp.float32),
            pltpu.VMEM((CHUNK, D), jnp.float32),
            pltpu.SemaphoreType.DMA,
            pltpu.SemaphoreType.DMA,
        ],
    )
    def gather_kernel(tbl_hbm, idx_hbm, out_hbm, idx_v, buf0, buf1, sem0, sem1):
        wid = lax.axis_index("s") * NC + lax.axis_index("c")
        base = wid * RPW
        pltpu.sync_copy(idx_hbm.at[pl.ds(base, RPW)], idx_v)
        bufs = (buf0, buf1)
        sems = (sem0, sem1)
        pending = pltpu.async_copy(
            tbl_hbm.at[idx_v.at[pl.ds(0, CHUNK)]], bufs[0], sems[0])
        for c in range(NCHUNK):
            nxt = None
            if c + 1 < NCHUNK:
                nxt = pltpu.async_copy(
                    tbl_hbm.at[idx_v.at[pl.ds((c + 1) * CHUNK, CHUNK)]],
                    bufs[(c + 1) % 2], sems[(c + 1) % 2])
            pending.wait()
            pltpu.sync_copy(bufs[c % 2],
                            out_hbm.at[pl.ds(base + c * CHUNK, CHUNK)])
            pending = nxt

    return gather_kernel(table_flat, flat_idx)


def _mlp_body(feat_ref, dense_ref, w1_ref, w2_ref, b_ref, out_ref):
    acc = jnp.dot(feat_ref[...], w1_ref[...], preferred_element_type=jnp.float32)
    acc = acc + jnp.dot(dense_ref[...], w2_ref[...],
                        preferred_element_type=jnp.float32)
    acc = acc + b_ref[...]
    out_ref[...] = jnp.maximum(acc, 0.0)


def _tc_mlp(feat, dense_p, w1, w2_p, b2d):
    bs = 1024
    grid = (B // bs,)
    return pl.pallas_call(
        _mlp_body,
        grid=grid,
        in_specs=[
            pl.BlockSpec((bs, F * D), lambda i: (i, 0)),
            pl.BlockSpec((bs, 16), lambda i: (i, 0)),
            pl.BlockSpec((F * D, OUT), lambda i: (0, 0)),
            pl.BlockSpec((16, OUT), lambda i: (0, 0)),
            pl.BlockSpec((1, OUT), lambda i: (0, 0)),
        ],
        out_specs=pl.BlockSpec((bs, OUT), lambda i: (i, 0)),
        out_shape=jax.ShapeDtypeStruct((B, OUT), jnp.float32),
        compiler_params=pltpu.CompilerParams(
            dimension_semantics=("arbitrary",),
        ),
    )(feat, dense_p, w1, w2_p, b2d)


def kernel(sparse_indices, dense_features, tables, W, b):
    flat_idx = (sparse_indices
                + (jnp.arange(F, dtype=jnp.int32) * V)[None, :]).reshape(-1)
    table_flat = tables.reshape(F * V, D)
    emb = _sc_gather(table_flat, flat_idx)
    feat = emb.reshape(B, F * D)
    dense_p = jnp.pad(dense_features, ((0, 0), (0, 16 - ND)))
    w1 = W[: F * D]
    w2_p = jnp.pad(W[F * D:], ((0, 16 - ND), (0, 0)))
    return _tc_mlp(feat, dense_p, w1, w2_p, b.reshape(1, OUT))


# speedup vs baseline: 8.1093x; 8.1093x over previous
import functools

import jax
import jax.numpy as jnp
from jax import lax
from jax.experimental import pallas as pl
from jax.experimental.pallas import tpu as pltpu
from jax.experimental.pallas import tpu_sc as plsc

B = 16384
F = 26
V = 100000
D = 32
ND = 13
OUT = 128
HIDDEN = F * D + ND

NC = 2
NS = 16
NW = NC * NS

ROWS = B * F
RPW = ROWS // NW
CHUNK = 1024
NCHUNK = RPW // CHUNK


def _sc_gather(table_flat, flat_idx):
    mesh = plsc.VectorSubcoreMesh(core_axis_name="c", subcore_axis_name="s",
                                  num_cores=NC, num_subcores=NS)

    @functools.partial(
        pl.kernel,
        out_type=jax.ShapeDtypeStruct((ROWS, D), jnp.float32),
        mesh=mesh,
        scratch_types=[
            pltpu.VMEM((RPW,), jnp.int32),
            pltpu.VMEM((CHUNK, D), jnp.float32),
            pltpu.VMEM((CHUNK, D), jnp.float32),
            pltpu.SemaphoreType.DMA,
            pltpu.SemaphoreType.DMA,
        ],
        compiler_params=pltpu.CompilerParams(use_tc_tiling_on_sc=False),
    )
    def gather_kernel(tbl_hbm, idx_hbm, out_hbm, idx_v, buf0, buf1, sem0, sem1):
        wid = lax.axis_index("s") * NC + lax.axis_index("c")
        base = wid * RPW
        pltpu.sync_copy(idx_hbm.at[pl.ds(base, RPW)], idx_v)
        bufs = (buf0, buf1)
        sems = (sem0, sem1)
        pending = pltpu.async_copy(
            tbl_hbm.at[idx_v.at[pl.ds(0, CHUNK)]], bufs[0], sems[0])
        for c in range(NCHUNK):
            nxt = None
            if c + 1 < NCHUNK:
                nxt = pltpu.async_copy(
                    tbl_hbm.at[idx_v.at[pl.ds((c + 1) * CHUNK, CHUNK)]],
                    bufs[(c + 1) % 2], sems[(c + 1) % 2])
            pending.wait()
            pltpu.sync_copy(bufs[c % 2],
                            out_hbm.at[pl.ds(base + c * CHUNK, CHUNK)])
            pending = nxt

    return gather_kernel(table_flat, flat_idx)


def _mlp_body(feat_ref, dense_ref, w1_ref, w2_ref, b_ref, out_ref):
    acc = jnp.dot(feat_ref[...], w1_ref[...], preferred_element_type=jnp.float32)
    acc = acc + jnp.dot(dense_ref[...], w2_ref[...],
                        preferred_element_type=jnp.float32)
    acc = acc + b_ref[...]
    out_ref[...] = jnp.maximum(acc, 0.0)


def _tc_mlp(feat, dense_p, w1, w2_p, b2d):
    bs = 1024
    grid = (B // bs,)
    return pl.pallas_call(
        _mlp_body,
        grid=grid,
        in_specs=[
            pl.BlockSpec((bs, F * D), lambda i: (i, 0)),
            pl.BlockSpec((bs, 16), lambda i: (i, 0)),
            pl.BlockSpec((F * D, OUT), lambda i: (0, 0)),
            pl.BlockSpec((16, OUT), lambda i: (0, 0)),
            pl.BlockSpec((1, OUT), lambda i: (0, 0)),
        ],
        out_specs=pl.BlockSpec((bs, OUT), lambda i: (i, 0)),
        out_shape=jax.ShapeDtypeStruct((B, OUT), jnp.float32),
        compiler_params=pltpu.CompilerParams(
            dimension_semantics=("arbitrary",),
        ),
    )(feat, dense_p, w1, w2_p, b2d)


def kernel(sparse_indices, dense_features, tables, W, b):
    flat_idx = (sparse_indices
                + (jnp.arange(F, dtype=jnp.int32) * V)[None, :]).reshape(-1)
    table_flat = tables.reshape(F * V, D)
    emb = _sc_gather(table_flat, flat_idx)
    feat = emb.reshape(B, F * D)
    dense_p = jnp.pad(dense_features, ((0, 0), (0, 16 - ND)))
    w1 = W[: F * D]
    w2_p = jnp.pad(W[F * D:], ((0, 16 - ND), (0, 0)))
    return _tc_mlp(feat, dense_p, w1, w2_p, b.reshape(1, OUT))


# speedup vs baseline: 25.7878x; 3.1800x over previous
import functools

import jax
import jax.numpy as jnp
from jax import lax
from jax.experimental import pallas as pl
from jax.experimental.pallas import tpu as pltpu
from jax.experimental.pallas import tpu_sc as plsc

B = 16384
F = 26
V = 100000
D = 32
ND = 13
OUT = 128
HID = F * D

NC = 2
NS = 16
NW = NC * NS
UPW = HID // NW
HALF = B // 2


def _sc_gather_t(tbl_fd, idx_t):
    mesh = plsc.VectorSubcoreMesh(core_axis_name="c", subcore_axis_name="s",
                                  num_cores=NC, num_subcores=NS)

    @functools.partial(
        pl.kernel,
        out_type=jax.ShapeDtypeStruct((HID, B), jnp.float32),
        mesh=mesh,
        scratch_types=[
            pltpu.VMEM((V,), jnp.float32),
            pltpu.VMEM((B,), jnp.int32),
            pltpu.VMEM((HALF,), jnp.float32),
        ],
        compiler_params=pltpu.CompilerParams(needs_layout_passes=False),
    )
    def gather_kernel(tbl_hbm, idx_hbm, out_hbm, stage_v, idx_v, out_v):
        wid = lax.axis_index("s") * NC + lax.axis_index("c")
        u0 = wid * UPW
        for k in range(UPW):
            uu = u0 + k
            f = uu // D
            pltpu.sync_copy(tbl_hbm.at[uu], stage_v)
            @pl.when(jnp.logical_or(k == 0, (uu % D) == 0))
            def _():
                pltpu.sync_copy(idx_hbm.at[f], idx_v)

            for half in range(2):
                def body(i, _):
                    vi = idx_v[pl.ds(half * HALF + i * 16, 16)]
                    vals = plsc.load_gather(stage_v, [vi])
                    out_v[pl.ds(i * 16, 16)] = vals
                    return 0
                lax.fori_loop(0, HALF // 16, body, 0)
                pltpu.sync_copy(out_v, out_hbm.at[uu, pl.ds(half * HALF, HALF)])

    return gather_kernel(tbl_fd, idx_t)


def _mlp_body(g_ref, dense_ref, w1_ref, w2_ref, b_ref, out_ref):
    for j in range(4):
        gj = g_ref[:, pl.ds(j * 128, 128)]
        acc = lax.dot_general(gj, w1_ref[...],
                              (((0,), (0,)), ((), ())),
                              preferred_element_type=jnp.float32)
        dj = dense_ref[pl.ds(j * 128, 128), :]
        acc = acc + jnp.dot(dj, w2_ref[...], preferred_element_type=jnp.float32)
        acc = acc + b_ref[...]
        out_ref[pl.ds(j * 128, 128), :] = jnp.maximum(acc, 0.0)


def _tc_mlp(g, dense_p, w1, w2_p, b2d):
    bs = 512
    grid = (B // bs,)
    return pl.pallas_call(
        _mlp_body,
        grid=grid,
        in_specs=[
            pl.BlockSpec((HID, bs), lambda i: (0, i)),
            pl.BlockSpec((bs, 16), lambda i: (i, 0)),
            pl.BlockSpec((HID, OUT), lambda i: (0, 0)),
            pl.BlockSpec((16, OUT), lambda i: (0, 0)),
            pl.BlockSpec((1, OUT), lambda i: (0, 0)),
        ],
        out_specs=pl.BlockSpec((bs, OUT), lambda i: (i, 0)),
        out_shape=jax.ShapeDtypeStruct((B, OUT), jnp.float32),
        compiler_params=pltpu.CompilerParams(
            dimension_semantics=("arbitrary",),
        ),
    )(g, dense_p, w1, w2_p, b2d)


def kernel(sparse_indices, dense_features, tables, W, b):
    tbl_fd = jnp.transpose(tables, (0, 2, 1)).reshape(HID, V)
    idx_t = jnp.transpose(sparse_indices, (1, 0))
    g = _sc_gather_t(tbl_fd, idx_t)
    dense_p = jnp.pad(dense_features, ((0, 0), (0, 16 - ND)))
    w1 = W[:HID]
    w2_p = jnp.pad(W[HID:], ((0, 16 - ND), (0, 0)))
    return _tc_mlp(g, dense_p, w1, w2_p, b.reshape(1, OUT))


# speedup vs baseline: 42.7628x; 1.6583x over previous
import functools

import jax
import jax.numpy as jnp
from jax import lax
from jax.experimental import pallas as pl
from jax.experimental.pallas import tpu as pltpu
from jax.experimental.pallas import tpu_sc as plsc

B = 16384
F = 26
V = 100000
D = 32
ND = 13
OUT = 128
HID = F * D

NC = 2
NS = 16
NW = NC * NS
UPW = HID // NW
BH = B // 2
VH0 = 49920
VH1 = V - VH0


def _sc_gather_t(tbl_fd, idx_t):
    mesh = plsc.VectorSubcoreMesh(core_axis_name="c", subcore_axis_name="s",
                                  num_cores=NC, num_subcores=NS)

    @functools.partial(
        pl.kernel,
        out_type=jax.ShapeDtypeStruct((HID, B), jnp.float32),
        mesh=mesh,
        scratch_types=[
            pltpu.VMEM((VH0,), jnp.float32),
            pltpu.VMEM((VH1,), jnp.float32),
            pltpu.VMEM((B,), jnp.int32),
            pltpu.VMEM((BH,), jnp.float32),
            pltpu.SemaphoreType.DMA,
            pltpu.SemaphoreType.DMA,
        ],
        compiler_params=pltpu.CompilerParams(needs_layout_passes=False),
    )
    def gather_kernel(tbl_hbm, idx_hbm, out_hbm, stage_a, stage_b, idx_v,
                      out_v, sem_a, sem_b):
        wid = lax.axis_index("s") * NC + lax.axis_index("c")
        u0 = wid * UPW

        def _pass(stage, h, bh, first):
            lo = 0 if h == 0 else VH0

            @plsc.parallel_loop(0, BH // 16, 1, unroll=8)
            def body(i):
                vi = idx_v[pl.ds(bh * BH + i * 16, 16)]
                if h == 0:
                    m = vi < VH0
                else:
                    m = vi >= VH0
                vl = jnp.where(m, vi - lo, 0)
                vals = plsc.load_gather(stage, [vl], mask=m)
                vals = jnp.where(m, vals, 0.0)
                if first:
                    out_v[pl.ds(i * 16, 16)] = vals
                else:
                    plsc.addupdate(out_v.at[pl.ds(i * 16, 16)], vals)

        pltpu.async_copy(tbl_hbm.at[u0, pl.ds(0, VH0)], stage_a, sem_a)
        pltpu.async_copy(tbl_hbm.at[u0, pl.ds(VH0, VH1)], stage_b, sem_b)

        def unit(k, _):
            uu = u0 + k

            @pl.when(jnp.logical_or(k == 0, (uu % D) == 0))
            def _():
                pltpu.sync_copy(idx_hbm.at[uu // D], idx_v)

            pltpu.make_async_copy(tbl_hbm.at[uu, pl.ds(0, VH0)],
                                  stage_a, sem_a).wait()
            _pass(stage_a, 0, 0, True)
            pltpu.make_async_copy(tbl_hbm.at[uu, pl.ds(VH0, VH1)],
                                  stage_b, sem_b).wait()
            _pass(stage_b, 1, 0, False)
            pltpu.sync_copy(out_v, out_hbm.at[uu, pl.ds(0, BH)])
            _pass(stage_a, 0, 1, True)

            @pl.when(k < UPW - 1)
            def _():
                pltpu.async_copy(tbl_hbm.at[uu + 1, pl.ds(0, VH0)],
                                 stage_a, sem_a)

            _pass(stage_b, 1, 1, False)
            pltpu.sync_copy(out_v, out_hbm.at[uu, pl.ds(BH, BH)])

            @pl.when(k < UPW - 1)
            def _():
                pltpu.async_copy(tbl_hbm.at[uu + 1, pl.ds(VH0, VH1)],
                                 stage_b, sem_b)

            return 0

        lax.fori_loop(0, UPW, unit, 0)

    return gather_kernel(tbl_fd, idx_t)


def _mlp_body(g_ref, dense_ref, w1_ref, w2_ref, b_ref, out_ref):
    for j in range(4):
        gj = g_ref[:, pl.ds(j * 128, 128)]
        acc = lax.dot_general(gj, w1_ref[...],
                              (((0,), (0,)), ((), ())),
                              preferred_element_type=jnp.float32)
        dj = dense_ref[pl.ds(j * 128, 128), :]
        acc = acc + jnp.dot(dj, w2_ref[...], preferred_element_type=jnp.float32)
        acc = acc + b_ref[...]
        out_ref[pl.ds(j * 128, 128), :] = jnp.maximum(acc, 0.0)


def _tc_mlp(g, dense_p, w1, w2_p, b2d):
    bs = 512
    grid = (B // bs,)
    return pl.pallas_call(
        _mlp_body,
        grid=grid,
        in_specs=[
            pl.BlockSpec((HID, bs), lambda i: (0, i)),
            pl.BlockSpec((bs, 16), lambda i: (i, 0)),
            pl.BlockSpec((HID, OUT), lambda i: (0, 0)),
            pl.BlockSpec((16, OUT), lambda i: (0, 0)),
            pl.BlockSpec((1, OUT), lambda i: (0, 0)),
        ],
        out_specs=pl.BlockSpec((bs, OUT), lambda i: (i, 0)),
        out_shape=jax.ShapeDtypeStruct((B, OUT), jnp.float32),
        compiler_params=pltpu.CompilerParams(
            dimension_semantics=("arbitrary",),
        ),
    )(g, dense_p, w1, w2_p, b2d)


def kernel(sparse_indices, dense_features, tables, W, b):
    tbl_fd = jnp.transpose(tables, (0, 2, 1)).reshape(HID, V)
    idx_t = jnp.transpose(sparse_indices, (1, 0))
    g = _sc_gather_t(tbl_fd, idx_t)
    dense_p = jnp.pad(dense_features, ((0, 0), (0, 16 - ND)))
    w1 = W[:HID]
    w2_p = jnp.pad(W[HID:], ((0, 16 - ND), (0, 0)))
    return _tc_mlp(g, dense_p, w1, w2_p, b.reshape(1, OUT))


# speedup vs baseline: 45.1552x; 1.0559x over previous
import functools

import jax
import jax.numpy as jnp
from jax import lax
from jax.experimental import pallas as pl
from jax.experimental.pallas import tpu as pltpu
from jax.experimental.pallas import tpu_sc as plsc

B = 16384
F = 26
V = 100000
D = 32
ND = 13
OUT = 128
HID = F * D

NC = 2
NS = 16
NW = NC * NS
UPW = HID // NW
BH = B // 2
VH0 = 49920
VH1 = V - VH0


def _sc_gather_t(tbl_fd, idx_t):
    mesh = plsc.VectorSubcoreMesh(core_axis_name="c", subcore_axis_name="s",
                                  num_cores=NC, num_subcores=NS)

    @functools.partial(
        pl.kernel,
        out_type=jax.ShapeDtypeStruct((HID, B), jnp.float32),
        mesh=mesh,
        scratch_types=[
            pltpu.VMEM((VH0,), jnp.float32),
            pltpu.VMEM((VH1,), jnp.float32),
            pltpu.VMEM((B,), jnp.int32),
            pltpu.VMEM((BH,), jnp.float32),
            pltpu.SemaphoreType.DMA,
            pltpu.SemaphoreType.DMA,
        ],
        compiler_params=pltpu.CompilerParams(needs_layout_passes=False),
    )
    def gather_kernel(tbl_hbm, idx_hbm, out_hbm, stage_a, stage_b, idx_v,
                      out_v, sem_a, sem_b):
        wid = lax.axis_index("s") * NC + lax.axis_index("c")
        u0 = wid * UPW

        def _pass(stage, h, bh, first):
            lo = 0 if h == 0 else VH0

            @plsc.parallel_loop(0, BH // 16, 1, unroll=8)
            def body(i):
                vi = idx_v[pl.ds(bh * BH + i * 16, 16)]
                if h == 0:
                    m = vi < VH0
                else:
                    m = vi >= VH0
                vl = jnp.where(m, vi - lo, 0)
                vals = plsc.load_gather(stage, [vl], mask=m)
                vals = jnp.where(m, vals, 0.0)
                if first:
                    out_v[pl.ds(i * 16, 16)] = vals
                else:
                    plsc.addupdate(out_v.at[pl.ds(i * 16, 16)], vals)

        pltpu.async_copy(tbl_hbm.at[u0, pl.ds(0, VH0)], stage_a, sem_a)
        pltpu.async_copy(tbl_hbm.at[u0, pl.ds(VH0, VH1)], stage_b, sem_b)

        def unit(k, _):
            uu = u0 + k

            @pl.when(jnp.logical_or(k == 0, (uu % D) == 0))
            def _():
                pltpu.sync_copy(idx_hbm.at[uu // D], idx_v)

            pltpu.make_async_copy(tbl_hbm.at[uu, pl.ds(0, VH0)],
                                  stage_a, sem_a).wait()
            _pass(stage_a, 0, 0, True)
            pltpu.make_async_copy(tbl_hbm.at[uu, pl.ds(VH0, VH1)],
                                  stage_b, sem_b).wait()
            _pass(stage_b, 1, 0, False)
            pltpu.sync_copy(out_v, out_hbm.at[uu, pl.ds(0, BH)])
            _pass(stage_a, 0, 1, True)

            @pl.when(k < UPW - 1)
            def _():
                pltpu.async_copy(tbl_hbm.at[uu + 1, pl.ds(0, VH0)],
                                 stage_a, sem_a)

            _pass(stage_b, 1, 1, False)
            pltpu.sync_copy(out_v, out_hbm.at[uu, pl.ds(BH, BH)])

            @pl.when(k < UPW - 1)
            def _():
                pltpu.async_copy(tbl_hbm.at[uu + 1, pl.ds(VH0, VH1)],
                                 stage_b, sem_b)

            return 0

        lax.fori_loop(0, UPW, unit, 0)

    return gather_kernel(tbl_fd, idx_t)


def _mlp_body(g_ref, dense_ref, w1_ref, w2_ref, b_ref, out_ref):
    for j in range(g_ref.shape[1] // 128):
        gj = g_ref[:, pl.ds(j * 128, 128)]
        acc = lax.dot_general(gj, w1_ref[...],
                              (((0,), (0,)), ((), ())),
                              preferred_element_type=jnp.float32)
        dj = dense_ref[pl.ds(j * 128, 128), :]
        acc = acc + jnp.dot(dj, w2_ref[...], preferred_element_type=jnp.float32)
        acc = acc + b_ref[...]
        out_ref[pl.ds(j * 128, 128), :] = jnp.maximum(acc, 0.0)


def _tc_mlp(g, dense_p, w1, w2_p, b2d):
    bs = 2048
    grid = (B // bs,)
    return pl.pallas_call(
        _mlp_body,
        grid=grid,
        in_specs=[
            pl.BlockSpec((HID, bs), lambda i: (0, i)),
            pl.BlockSpec((bs, 16), lambda i: (i, 0)),
            pl.BlockSpec((HID, OUT), lambda i: (0, 0)),
            pl.BlockSpec((16, OUT), lambda i: (0, 0)),
            pl.BlockSpec((1, OUT), lambda i: (0, 0)),
        ],
        out_specs=pl.BlockSpec((bs, OUT), lambda i: (i, 0)),
        out_shape=jax.ShapeDtypeStruct((B, OUT), jnp.float32),
        compiler_params=pltpu.CompilerParams(
            dimension_semantics=("arbitrary",),
        ),
    )(g, dense_p, w1, w2_p, b2d)


def kernel(sparse_indices, dense_features, tables, W, b):
    tbl_fd = jnp.transpose(tables, (0, 2, 1)).reshape(HID, V)
    idx_t = jnp.transpose(sparse_indices, (1, 0))
    g = _sc_gather_t(tbl_fd, idx_t)
    dense_p = jnp.pad(dense_features, ((0, 0), (0, 16 - ND)))
    w1 = W[:HID]
    w2_p = jnp.pad(W[HID:], ((0, 16 - ND), (0, 0)))
    return _tc_mlp(g, dense_p, w1, w2_p, b.reshape(1, OUT))


# speedup vs baseline: 45.3694x; 1.0047x over previous
import functools

import jax
import jax.numpy as jnp
from jax import lax
from jax.experimental import pallas as pl
from jax.experimental.pallas import tpu as pltpu
from jax.experimental.pallas import tpu_sc as plsc

B = 16384
F = 26
V = 100000
D = 32
ND = 13
OUT = 128
HID = F * D

NC = 2
NS = 16
NW = NC * NS
UPW = HID // NW
BH = B // 2
VH0 = 49920
VH1 = V - VH0


def _sc_gather_t(tbl_fd, idx_t):
    mesh = plsc.VectorSubcoreMesh(core_axis_name="c", subcore_axis_name="s",
                                  num_cores=NC, num_subcores=NS)

    @functools.partial(
        pl.kernel,
        out_type=jax.ShapeDtypeStruct((HID, B), jnp.float32),
        mesh=mesh,
        scratch_types=[
            pltpu.VMEM((VH0,), jnp.float32),
            pltpu.VMEM((VH1,), jnp.float32),
            pltpu.VMEM((B,), jnp.int32),
            pltpu.VMEM((BH,), jnp.float32),
            pltpu.SemaphoreType.DMA,
            pltpu.SemaphoreType.DMA,
        ],
        compiler_params=pltpu.CompilerParams(needs_layout_passes=False),
    )
    def gather_kernel(tbl_hbm, idx_hbm, out_hbm, stage_a, stage_b, idx_v,
                      out_v, sem_a, sem_b):
        wid = lax.axis_index("s") * NC + lax.axis_index("c")
        u0 = wid * UPW

        def _pass(stage, h, bh, first):
            lo = 0 if h == 0 else VH0

            @plsc.parallel_loop(0, BH // 16, 1, unroll=8)
            def body(i):
                vi = idx_v[pl.ds(bh * BH + i * 16, 16)]
                if h == 0:
                    m = vi < VH0
                else:
                    m = vi >= VH0
                vl = jnp.where(m, vi - lo, 0)
                vals = plsc.load_gather(stage, [vl], mask=m)
                vals = jnp.where(m, vals, 0.0)
                if first:
                    out_v[pl.ds(i * 16, 16)] = vals
                else:
                    plsc.addupdate(out_v.at[pl.ds(i * 16, 16)], vals)

        pltpu.async_copy(tbl_hbm.at[u0, pl.ds(0, VH0)], stage_a, sem_a)
        pltpu.async_copy(tbl_hbm.at[u0, pl.ds(VH0, VH1)], stage_b, sem_b)

        def unit(k, _):
            uu = u0 + k

            @pl.when(jnp.logical_or(k == 0, (uu % D) == 0))
            def _():
                pltpu.sync_copy(idx_hbm.at[uu // D], idx_v)

            pltpu.make_async_copy(tbl_hbm.at[uu, pl.ds(0, VH0)],
                                  stage_a, sem_a).wait()
            _pass(stage_a, 0, 0, True)
            pltpu.make_async_copy(tbl_hbm.at[uu, pl.ds(VH0, VH1)],
                                  stage_b, sem_b).wait()
            _pass(stage_b, 1, 0, False)
            pltpu.sync_copy(out_v, out_hbm.at[uu, pl.ds(0, BH)])
            _pass(stage_a, 0, 1, True)

            @pl.when(k < UPW - 1)
            def _():
                pltpu.async_copy(tbl_hbm.at[uu + 1, pl.ds(0, VH0)],
                                 stage_a, sem_a)

            _pass(stage_b, 1, 1, False)
            pltpu.sync_copy(out_v, out_hbm.at[uu, pl.ds(BH, BH)])

            @pl.when(k < UPW - 1)
            def _():
                pltpu.async_copy(tbl_hbm.at[uu + 1, pl.ds(VH0, VH1)],
                                 stage_b, sem_b)

            return 0

        lax.fori_loop(0, UPW, unit, 0)

    return gather_kernel(tbl_fd, idx_t)


def _mlp_body(g_ref, dense_ref, w1_ref, w2_ref, b_ref, out_ref):
    acc = lax.dot_general(g_ref[...], w1_ref[...],
                          (((0,), (0,)), ((), ())),
                          preferred_element_type=jnp.float32)
    acc = acc + jnp.dot(dense_ref[...], w2_ref[...],
                        preferred_element_type=jnp.float32)
    acc = acc + b_ref[...]
    out_ref[...] = jnp.maximum(acc, 0.0)


def _tc_mlp(g, dense_p, w1, w2_p, b2d):
    bs = 2048
    grid = (B // bs,)
    return pl.pallas_call(
        _mlp_body,
        grid=grid,
        in_specs=[
            pl.BlockSpec((HID, bs), lambda i: (0, i)),
            pl.BlockSpec((bs, 16), lambda i: (i, 0)),
            pl.BlockSpec((HID, OUT), lambda i: (0, 0)),
            pl.BlockSpec((16, OUT), lambda i: (0, 0)),
            pl.BlockSpec((1, OUT), lambda i: (0, 0)),
        ],
        out_specs=pl.BlockSpec((bs, OUT), lambda i: (i, 0)),
        out_shape=jax.ShapeDtypeStruct((B, OUT), jnp.float32),
        compiler_params=pltpu.CompilerParams(
            dimension_semantics=("arbitrary",),
        ),
    )(g, dense_p, w1, w2_p, b2d)


def kernel(sparse_indices, dense_features, tables, W, b):
    tbl_fd = jnp.transpose(tables, (0, 2, 1)).reshape(HID, V)
    idx_t = jnp.transpose(sparse_indices, (1, 0))
    g = _sc_gather_t(tbl_fd, idx_t)
    dense_p = jnp.pad(dense_features, ((0, 0), (0, 16 - ND)))
    w1 = W[:HID]
    w2_p = jnp.pad(W[HID:], ((0, 16 - ND), (0, 0)))
    return _tc_mlp(g, dense_p, w1, w2_p, b.reshape(1, OUT))
